# baseline (speedup 1.0000x reference)
import functools
import jax
import jax.numpy as jnp
from jax import lax
from jax.experimental import pallas as pl
from jax.experimental.pallas import tpu as pltpu
from jax.experimental.pallas import tpu_sc as plsc

N = 10000
D = 256
Q = 64
E = 160000
EPS = 1e-7

NS = 16
CHUNK = 128
NCHUNK = 80
NBUF = 2
NROUND = NCHUNK // (2 * NBUF)
E_PAD = NS * NCHUNK * CHUNK
TRASH = N
NPAD = 10112
RPW = NPAD // NS

ROW_BLK = 2000
CMB_BLK = 400


def _logmap_body(x_ref, q0_ref, q1_ref, q2_ref, q3_ref):
    x = x_ref[...]
    nrm = jnp.sqrt(jnp.sum(x * x, axis=1, keepdims=True))
    nrm = jnp.maximum(nrm, EPS)
    y = jnp.minimum(nrm, 1.0 - 1e-5)
    scale = 0.5 * jnp.log((1.0 + y) / (1.0 - y)) / nrm
    xt = x * scale
    def pack(q):
        v = xt[:, q * Q:(q + 1) * Q].reshape(ROW_BLK // 2, 2, Q)
        return jnp.concatenate([v[:, 0, :], v[:, 1, :]], axis=1)

    q0_ref[...] = pack(0)
    q1_ref[...] = pack(1)
    q2_ref[...] = pack(2)
    q3_ref[...] = pack(3)


_QP_SPEC = pl.BlockSpec((ROW_BLK // 2, 2 * Q), lambda i: (i, 0))
_QP_SHAPE = jax.ShapeDtypeStruct((N // 2, 2 * Q), jnp.float32)

_logmap = pl.pallas_call(
    _logmap_body,
    grid=(N // ROW_BLK,),
    in_specs=[pl.BlockSpec((ROW_BLK, D), lambda i: (i, 0))],
    out_specs=[_QP_SPEC, _QP_SPEC, _QP_SPEC, _QP_SPEC],
    out_shape=[_QP_SHAPE, _QP_SHAPE, _QP_SHAPE, _QP_SHAPE],
)


@functools.partial(
    pl.kernel,
    mesh=plsc.VectorSubcoreMesh(core_axis_name="c", subcore_axis_name="s"),
    compiler_params=pltpu.CompilerParams(use_tc_tiling_on_sc=False),
    out_type=[
        jax.ShapeDtypeStruct((NPAD, Q), jnp.float32),
        jax.ShapeDtypeStruct((NPAD, Q), jnp.float32),
        jax.ShapeDtypeStruct((NPAD, Q), jnp.float32),
        jax.ShapeDtypeStruct((NPAD, Q), jnp.float32),
        jax.ShapeDtypeStruct((NPAD, 16), jnp.float32),
        jax.ShapeDtypeStruct((NPAD, 16), jnp.float32),
    ],
    scratch_types=[
        pltpu.VMEM((NCHUNK, CHUNK), jnp.int32),
        pltpu.VMEM((NCHUNK, CHUNK), jnp.int32),
        pltpu.VMEM((CHUNK, Q), jnp.float32),
        pltpu.VMEM((CHUNK, Q), jnp.float32),
        pltpu.VMEM((CHUNK, Q), jnp.float32),
        pltpu.VMEM((CHUNK, Q), jnp.float32),
        pltpu.VMEM((CHUNK, 16), jnp.float32),
        pltpu.VMEM_SHARED((NPAD, Q), jnp.float32),
        pltpu.VMEM_SHARED((NPAD, 16), jnp.float32),
    ] + [pltpu.SemaphoreType.DMA] * 9,
)
def _sc_agg(xq0_hbm, xq1_hbm, xq2_hbm, xq3_hbm, src_hbm, dst_hbm,
            zrow_hbm, zdeg_hbm, ones_hbm,
            agg0_hbm, agg1_hbm, agg2_hbm, agg3_hbm, dega_hbm, degb_hbm,
            src_v, dst_v, a0, a1, b0, b1, ones_v,
            acc_sh, deg_sh,
            ga0, ga1, gb0, gb1,
            sa0, sa1, sb0, sb1, dsem):
    c = lax.axis_index("c")
    s = lax.axis_index("s")
    rows = pl.ds(s * RPW, RPW)
    bufs = [[a0, a1], [b0, b1]]
    gsems = [[ga0, ga1], [gb0, gb1]]
    ssems = [[sa0, sa1], [sb0, sb1]]

    pltpu.sync_copy(ones_hbm, ones_v)

    def one_pass(xq_hbm, agg_hbm, deg_half, deg_hbm):
        pltpu.sync_copy(src_hbm.at[s], src_v)
        pltpu.sync_copy(dst_hbm.at[s], dst_v)
        pltpu.sync_copy(zrow_hbm, acc_sh.at[rows])
        if deg_half is not None:
            pltpu.sync_copy(zdeg_hbm, deg_sh.at[rows])
        plsc.subcore_barrier()

        hdma = xq_hbm.at[pl.ds(0, CHUNK)]

        for h in range(2):
            for b in range(NBUF):
                pltpu.async_copy(
                    xq_hbm.at[src_v.at[h * NBUF + b]], bufs[h][b],
                    gsems[h][b])

        def scat_half(t, h, base):
            for b in range(NBUF):
                pltpu.make_async_copy(hdma, bufs[h][b], gsems[h][b]).wait()
                pltpu.async_copy(
                    bufs[h][b], acc_sh.at[dst_v.at[base + b]], ssems[h][b],
                    add=True)
            if deg_half is not None:
                @pl.when(deg_half(t))
                def _():
                    for b in range(NBUF):
                        pltpu.async_copy(
                            ones_v, deg_sh.at[dst_v.at[base + b]], dsem,
                            add=True)

        def refill_half(t, h, base):
            for b in range(NBUF):
                pltpu.make_async_copy(hdma, bufs[h][b], ssems[h][b]).wait()

                @pl.when(t < NROUND - 1)
                def _(h=h, b=b, base=base):
                    pltpu.async_copy(
                        xq_hbm.at[src_v.at[base + b]], bufs[h][b],
                        gsems[h][b])

        def rnd(t, carry):
            base = t * 2 * NBUF
            scat_half(t, 0, base)
            scat_half(t, 1, base + NBUF)
            refill_half(t, 0, base + 2 * NBUF)
            refill_half(t, 1, base + 3 * NBUF)
            if deg_half is not None:
                @pl.when(deg_half(t))
                def _():
                    for _b in range(2 * NBUF):
                        pltpu.make_async_copy(ones_hbm, ones_v, dsem).wait()
            return carry

        lax.fori_loop(0, NROUND, rnd, 0)
        plsc.subcore_barrier()
        pltpu.sync_copy(acc_sh.at[rows], agg_hbm.at[rows])
        if deg_half is not None:
            pltpu.sync_copy(deg_sh.at[rows], deg_hbm.at[rows])

    @pl.when(c == 0)
    def _():
        one_pass(xq0_hbm, agg0_hbm, lambda t: t < NROUND // 2, dega_hbm)
        one_pass(xq1_hbm, agg1_hbm, None, None)

    @pl.when(c == 1)
    def _():
        one_pass(xq2_hbm, agg2_hbm, lambda t: t >= NROUND // 2, degb_hbm)
        one_pass(xq3_hbm, agg3_hbm, None, None)


def _combine_body(q0_ref, q1_ref, q2_ref, q3_ref,
                  a0_ref, a1_ref, a2_ref, a3_ref, dega_ref, degb_ref,
                  ws_ref, wn_ref, b_ref, o_ref):
    def unpack(r):
        p = r[...]
        v = jnp.stack([p[:, :Q], p[:, Q:]], axis=1)
        return v.reshape(CMB_BLK, Q)

    deg = dega_ref[:, 0:1] + degb_ref[:, 0:1]
    inv = 1.0 / jnp.maximum(deg, 1.0)
    xt = jnp.concatenate(
        [unpack(q0_ref), unpack(q1_ref), unpack(q2_ref), unpack(q3_ref)],
        axis=1)
    ag = jnp.concatenate(
        [unpack(a0_ref), unpack(a1_ref), unpack(a2_ref), unpack(a3_ref)],
        axis=1)
    dn = (((1,), (1,)), ((), ()))
    h = (lax.dot_general(xt, ws_ref[...], dn,
                         preferred_element_type=jnp.float32)
         + lax.dot_general(ag * inv, wn_ref[...], dn,
                           preferred_element_type=jnp.float32)
         + b_ref[...])
    nrm = jnp.sqrt(jnp.sum(h * h, axis=1, keepdims=True))
    nrm = jnp.maximum(nrm, EPS)
    o_ref[...] = jnp.tanh(nrm) * h / nrm


_W_SPEC = pl.BlockSpec((D, D), lambda i: (0, 0))
_DEG_SPEC = pl.BlockSpec((CMB_BLK, 16), lambda i: (i, 0))
_CP_SPEC = pl.BlockSpec((CMB_BLK // 2, 2 * Q), lambda i: (i, 0))

_combine = pl.pallas_call(
    _combine_body,
    grid=(N // CMB_BLK,),
    in_specs=[
        _CP_SPEC, _CP_SPEC, _CP_SPEC, _CP_SPEC,
        _CP_SPEC, _CP_SPEC, _CP_SPEC, _CP_SPEC,
        _DEG_SPEC, _DEG_SPEC,
        _W_SPEC, _W_SPEC,
        pl.BlockSpec((1, D), lambda i: (0, 0)),
    ],
    out_specs=pl.BlockSpec((CMB_BLK, D), lambda i: (i, 0)),
    out_shape=jax.ShapeDtypeStruct((N, D), jnp.float32),
)


def kernel(x, edge_index, W_self, b_self, W_neigh, b_neigh):
    src = edge_index[0].astype(jnp.int32)
    dst = edge_index[1].astype(jnp.int32)
    pad = E_PAD - E
    src2 = jnp.concatenate([src, jnp.zeros((pad,), jnp.int32)]).reshape(
        NS, NCHUNK, CHUNK)
    dst2 = jnp.concatenate([dst, jnp.full((pad,), TRASH, jnp.int32)]).reshape(
        NS, NCHUNK, CHUNK)

    q0p, q1p, q2p, q3p = _logmap(x)
    q0, q1, q2, q3 = (q.reshape(N, Q) for q in (q0p, q1p, q2p, q3p))

    zrow = jnp.zeros((RPW, Q), jnp.float32)
    zdeg = jnp.zeros((RPW, 16), jnp.float32)
    ones = jnp.ones((CHUNK, 16), jnp.float32)
    a0, a1, a2, a3, dega, degb = _sc_agg(
        q0, q1, q2, q3, src2, dst2, zrow, zdeg, ones)

    a0p, a1p, a2p, a3p = (
        a.reshape(NPAD // 2, 2 * Q) for a in (a0, a1, a2, a3))
    bias = (b_self + b_neigh).reshape(1, D)
    return _combine(q0p, q1p, q2p, q3p, a0p, a1p, a2p, a3p, dega, degb,
                    W_self, W_neigh, bias)


# speedup vs baseline: 1.0977x; 1.0977x over previous
import functools
import jax
import jax.numpy as jnp
from jax import lax
from jax.experimental import pallas as pl
from jax.experimental.pallas import tpu as pltpu
from jax.experimental.pallas import tpu_sc as plsc

N = 10000
D = 256
Q = 64
E = 160000
EPS = 1e-7

NS = 16
CHUNK = 64
NCHUNK = 160
NBUF = 4
NROUND = NCHUNK // (2 * NBUF)
E_PAD = NS * NCHUNK * CHUNK
TRASH = N
NPAD = 10112
RPW = NPAD // NS

ROW_BLK = 2000
CMB_BLK = 400


def _logmap_body(x_ref, q0_ref, q1_ref, q2_ref, q3_ref):
    x = x_ref[...]
    nrm = jnp.sqrt(jnp.sum(x * x, axis=1, keepdims=True))
    nrm = jnp.maximum(nrm, EPS)
    y = jnp.minimum(nrm, 1.0 - 1e-5)
    scale = 0.5 * jnp.log((1.0 + y) / (1.0 - y)) / nrm
    xt = x * scale
    def pack(q):
        v = xt[:, q * Q:(q + 1) * Q].reshape(ROW_BLK // 2, 2, Q)
        return jnp.concatenate([v[:, 0, :], v[:, 1, :]], axis=1)

    q0_ref[...] = pack(0)
    q1_ref[...] = pack(1)
    q2_ref[...] = pack(2)
    q3_ref[...] = pack(3)


_QP_SPEC = pl.BlockSpec((ROW_BLK // 2, 2 * Q), lambda i: (i, 0))
_QP_SHAPE = jax.ShapeDtypeStruct((N // 2, 2 * Q), jnp.float32)

_logmap = pl.pallas_call(
    _logmap_body,
    grid=(N // ROW_BLK,),
    in_specs=[pl.BlockSpec((ROW_BLK, D), lambda i: (i, 0))],
    out_specs=[_QP_SPEC, _QP_SPEC, _QP_SPEC, _QP_SPEC],
    out_shape=[_QP_SHAPE, _QP_SHAPE, _QP_SHAPE, _QP_SHAPE],
)


@functools.partial(
    pl.kernel,
    mesh=plsc.VectorSubcoreMesh(core_axis_name="c", subcore_axis_name="s"),
    compiler_params=pltpu.CompilerParams(use_tc_tiling_on_sc=False),
    out_type=[
        jax.ShapeDtypeStruct((NPAD, Q), jnp.float32),
        jax.ShapeDtypeStruct((NPAD, Q), jnp.float32),
        jax.ShapeDtypeStruct((NPAD, Q), jnp.float32),
        jax.ShapeDtypeStruct((NPAD, Q), jnp.float32),
        jax.ShapeDtypeStruct((NPAD, 16), jnp.float32),
        jax.ShapeDtypeStruct((NPAD, 16), jnp.float32),
    ],
    scratch_types=[
        pltpu.VMEM((NCHUNK, CHUNK), jnp.int32),
        pltpu.VMEM((NCHUNK, CHUNK), jnp.int32),
        pltpu.VMEM((CHUNK, Q), jnp.float32),
        pltpu.VMEM((CHUNK, Q), jnp.float32),
        pltpu.VMEM((CHUNK, Q), jnp.float32),
        pltpu.VMEM((CHUNK, Q), jnp.float32),
        pltpu.VMEM((CHUNK, Q), jnp.float32),
        pltpu.VMEM((CHUNK, Q), jnp.float32),
        pltpu.VMEM((CHUNK, Q), jnp.float32),
        pltpu.VMEM((CHUNK, Q), jnp.float32),
        pltpu.VMEM((CHUNK, 16), jnp.float32),
        pltpu.VMEM_SHARED((NPAD, Q), jnp.float32),
        pltpu.VMEM_SHARED((NPAD, 16), jnp.float32),
    ] + [pltpu.SemaphoreType.DMA] * 17,
)
def _sc_agg(xq0_hbm, xq1_hbm, xq2_hbm, xq3_hbm, src_hbm, dst_hbm,
            zrow_hbm, zdeg_hbm, ones_hbm,
            agg0_hbm, agg1_hbm, agg2_hbm, agg3_hbm, dega_hbm, degb_hbm,
            src_v, dst_v, a0, a1, a2, a3, b0, b1, b2, b3, ones_v,
            acc_sh, deg_sh,
            ga0, ga1, ga2, ga3, gb0, gb1, gb2, gb3,
            sa0, sa1, sa2, sa3, sb0, sb1, sb2, sb3, dsem):
    c = lax.axis_index("c")
    s = lax.axis_index("s")
    rows = pl.ds(s * RPW, RPW)
    bufs = [[a0, a1, a2, a3], [b0, b1, b2, b3]]
    gsems = [[ga0, ga1, ga2, ga3], [gb0, gb1, gb2, gb3]]
    ssems = [[sa0, sa1, sa2, sa3], [sb0, sb1, sb2, sb3]]

    pltpu.sync_copy(ones_hbm, ones_v)

    def one_pass(xq_hbm, agg_hbm, deg_half, deg_hbm):
        pltpu.sync_copy(src_hbm.at[s], src_v)
        pltpu.sync_copy(dst_hbm.at[s], dst_v)
        pltpu.sync_copy(zrow_hbm, acc_sh.at[rows])
        if deg_half is not None:
            pltpu.sync_copy(zdeg_hbm, deg_sh.at[rows])
        plsc.subcore_barrier()

        hdma = xq_hbm.at[pl.ds(0, CHUNK)]

        for h in range(2):
            for b in range(NBUF):
                pltpu.async_copy(
                    xq_hbm.at[src_v.at[h * NBUF + b]], bufs[h][b],
                    gsems[h][b])

        def scat_half(t, h, base):
            for b in range(NBUF):
                pltpu.make_async_copy(hdma, bufs[h][b], gsems[h][b]).wait()
                pltpu.async_copy(
                    bufs[h][b], acc_sh.at[dst_v.at[base + b]], ssems[h][b],
                    add=True)
            if deg_half is not None:
                @pl.when(deg_half(t))
                def _():
                    for b in range(NBUF):
                        pltpu.async_copy(
                            ones_v, deg_sh.at[dst_v.at[base + b]], dsem,
                            add=True)

        def refill_half(t, h, base):
            for b in range(NBUF):
                pltpu.make_async_copy(hdma, bufs[h][b], ssems[h][b]).wait()

                @pl.when(t < NROUND - 1)
                def _(h=h, b=b, base=base):
                    pltpu.async_copy(
                        xq_hbm.at[src_v.at[base + b]], bufs[h][b],
                        gsems[h][b])

        def rnd(t, carry):
            base = t * 2 * NBUF
            scat_half(t, 0, base)
            scat_half(t, 1, base + NBUF)
            refill_half(t, 0, base + 2 * NBUF)
            refill_half(t, 1, base + 3 * NBUF)
            if deg_half is not None:
                @pl.when(deg_half(t))
                def _():
                    for _b in range(2 * NBUF):
                        pltpu.make_async_copy(ones_hbm, ones_v, dsem).wait()
            return carry

        lax.fori_loop(0, NROUND, rnd, 0)
        plsc.subcore_barrier()
        pltpu.sync_copy(acc_sh.at[rows], agg_hbm.at[rows])
        if deg_half is not None:
            pltpu.sync_copy(deg_sh.at[rows], deg_hbm.at[rows])

    @pl.when(c == 0)
    def _():
        one_pass(xq0_hbm, agg0_hbm, lambda t: t < NROUND // 2, dega_hbm)
        one_pass(xq1_hbm, agg1_hbm, None, None)

    @pl.when(c == 1)
    def _():
        one_pass(xq2_hbm, agg2_hbm, lambda t: t >= NROUND // 2, degb_hbm)
        one_pass(xq3_hbm, agg3_hbm, None, None)


def _combine_body(q0_ref, q1_ref, q2_ref, q3_ref,
                  a0_ref, a1_ref, a2_ref, a3_ref, dega_ref, degb_ref,
                  w2s_ref, w2n_ref, b_ref, o_ref):
    B2 = CMB_BLK // 2
    xt = jnp.concatenate(
        [q0_ref[...], q1_ref[...], q2_ref[...], q3_ref[...]], axis=1)
    ag = jnp.concatenate(
        [a0_ref[...], a1_ref[...], a2_ref[...], a3_ref[...]], axis=1)
    deg = (dega_ref[:, 0:1] + degb_ref[:, 0:1]).reshape(B2, 2, 1)
    inv = 1.0 / jnp.maximum(deg, 1.0)
    iv0 = jnp.broadcast_to(inv[:, 0, :], (B2, Q))
    iv1 = jnp.broadcast_to(inv[:, 1, :], (B2, Q))
    ivrow = jnp.concatenate([iv0, iv1] * 4, axis=1)
    h2 = (jnp.dot(xt, w2s_ref[...], preferred_element_type=jnp.float32)
          + jnp.dot(ag * ivrow, w2n_ref[...],
                    preferred_element_type=jnp.float32)
          + b_ref[...])
    he = h2[:, :D]
    ho = h2[:, D:]

    def emap(h):
        nrm = jnp.sqrt(jnp.sum(h * h, axis=1, keepdims=True))
        nrm = jnp.maximum(nrm, EPS)
        return jnp.tanh(nrm) * h / nrm

    oe = emap(he)
    oo = emap(ho)
    o_ref[...] = jnp.stack([oe, oo], axis=1).reshape(CMB_BLK, D)


_W_SPEC = pl.BlockSpec((2 * D, 2 * D), lambda i: (0, 0))
_DEG_SPEC = pl.BlockSpec((CMB_BLK, 16), lambda i: (i, 0))
_CP_SPEC = pl.BlockSpec((CMB_BLK // 2, 2 * Q), lambda i: (i, 0))

_combine = pl.pallas_call(
    _combine_body,
    grid=(N // CMB_BLK,),
    in_specs=[
        _CP_SPEC, _CP_SPEC, _CP_SPEC, _CP_SPEC,
        _CP_SPEC, _CP_SPEC, _CP_SPEC, _CP_SPEC,
        _DEG_SPEC, _DEG_SPEC,
        _W_SPEC, _W_SPEC,
        pl.BlockSpec((1, 2 * D), lambda i: (0, 0)),
    ],
    out_specs=pl.BlockSpec((CMB_BLK, D), lambda i: (i, 0)),
    out_shape=jax.ShapeDtypeStruct((N, D), jnp.float32),
)


def kernel(x, edge_index, W_self, b_self, W_neigh, b_neigh):
    src = edge_index[0].astype(jnp.int32)
    dst = edge_index[1].astype(jnp.int32)
    pad = E_PAD - E
    src2 = jnp.concatenate([src, jnp.zeros((pad,), jnp.int32)]).reshape(
        NS, NCHUNK, CHUNK)
    dst2 = jnp.concatenate([dst, jnp.full((pad,), TRASH, jnp.int32)]).reshape(
        NS, NCHUNK, CHUNK)

    q0p, q1p, q2p, q3p = _logmap(x)
    q0, q1, q2, q3 = (q.reshape(N, Q) for q in (q0p, q1p, q2p, q3p))

    zrow = jnp.zeros((RPW, Q), jnp.float32)
    zdeg = jnp.zeros((RPW, 16), jnp.float32)
    ones = jnp.ones((CHUNK, 16), jnp.float32)
    a0, a1, a2, a3, dega, degb = _sc_agg(
        q0, q1, q2, q3, src2, dst2, zrow, zdeg, ones)

    a0p, a1p, a2p, a3p = (
        a.reshape(NPAD // 2, 2 * Q) for a in (a0, a1, a2, a3))

    def blockdiag(w):
        z = jnp.zeros((Q, D), jnp.float32)
        blks = []
        for q in range(4):
            wq = w[:, q * Q:(q + 1) * Q].T
            blks.append(jnp.concatenate([wq, z], axis=1))
            blks.append(jnp.concatenate([z, wq], axis=1))
        return jnp.concatenate(blks, axis=0)

    w2s = blockdiag(W_self)
    w2n = blockdiag(W_neigh)
    bias = jnp.tile((b_self + b_neigh), 2).reshape(1, 2 * D)
    return _combine(q0p, q1p, q2p, q3p, a0p, a1p, a2p, a3p, dega, degb,
                    w2s, w2n, bias)


# speedup vs baseline: 1.1401x; 1.0386x over previous
import functools
import jax
import jax.numpy as jnp
from jax import lax
from jax.experimental import pallas as pl
from jax.experimental.pallas import tpu as pltpu
from jax.experimental.pallas import tpu_sc as plsc

N = 10000
D = 256
Q = 64
E = 160000
EPS = 1e-7

NS = 16
CHUNK = 64
NCHUNK = 160
NBUF = 4
NROUND = NCHUNK // (2 * NBUF)
E_PAD = NS * NCHUNK * CHUNK
TRASH = N
NPAD = 10112
RPW = NPAD // NS

ROW_BLK = 2000
CMB_BLK = 2000


def _logmap_body(x_ref, q0_ref, q1_ref, q2_ref, q3_ref):
    x = x_ref[...]
    nrm = jnp.sqrt(jnp.sum(x * x, axis=1, keepdims=True))
    nrm = jnp.maximum(nrm, EPS)
    y = jnp.minimum(nrm, 1.0 - 1e-5)
    scale = 0.5 * jnp.log((1.0 + y) / (1.0 - y)) / nrm
    xt = x * scale
    def pack(q):
        v = xt[:, q * Q:(q + 1) * Q].reshape(ROW_BLK // 2, 2, Q)
        return jnp.concatenate([v[:, 0, :], v[:, 1, :]], axis=1)

    q0_ref[...] = pack(0)
    q1_ref[...] = pack(1)
    q2_ref[...] = pack(2)
    q3_ref[...] = pack(3)


_QP_SPEC = pl.BlockSpec((ROW_BLK // 2, 2 * Q), lambda i: (i, 0))
_QP_SHAPE = jax.ShapeDtypeStruct((N // 2, 2 * Q), jnp.float32)

_logmap = pl.pallas_call(
    _logmap_body,
    grid=(N // ROW_BLK,),
    in_specs=[pl.BlockSpec((ROW_BLK, D), lambda i: (i, 0))],
    out_specs=[_QP_SPEC, _QP_SPEC, _QP_SPEC, _QP_SPEC],
    out_shape=[_QP_SHAPE, _QP_SHAPE, _QP_SHAPE, _QP_SHAPE],
)


@functools.partial(
    pl.kernel,
    mesh=plsc.VectorSubcoreMesh(core_axis_name="c", subcore_axis_name="s"),
    compiler_params=pltpu.CompilerParams(use_tc_tiling_on_sc=False),
    out_type=[
        jax.ShapeDtypeStruct((NPAD, Q), jnp.float32),
        jax.ShapeDtypeStruct((NPAD, Q), jnp.float32),
        jax.ShapeDtypeStruct((NPAD, Q), jnp.float32),
        jax.ShapeDtypeStruct((NPAD, Q), jnp.float32),
        jax.ShapeDtypeStruct((NPAD, 16), jnp.float32),
        jax.ShapeDtypeStruct((NPAD, 16), jnp.float32),
    ],
    scratch_types=[
        pltpu.VMEM((NCHUNK, CHUNK), jnp.int32),
        pltpu.VMEM((NCHUNK, CHUNK), jnp.int32),
        pltpu.VMEM((CHUNK, Q), jnp.float32),
        pltpu.VMEM((CHUNK, Q), jnp.float32),
        pltpu.VMEM((CHUNK, Q), jnp.float32),
        pltpu.VMEM((CHUNK, Q), jnp.float32),
        pltpu.VMEM((CHUNK, Q), jnp.float32),
        pltpu.VMEM((CHUNK, Q), jnp.float32),
        pltpu.VMEM((CHUNK, Q), jnp.float32),
        pltpu.VMEM((CHUNK, Q), jnp.float32),
        pltpu.VMEM((CHUNK, 16), jnp.float32),
        pltpu.VMEM_SHARED((NPAD, Q), jnp.float32),
        pltpu.VMEM_SHARED((NPAD, 16), jnp.float32),
    ] + [pltpu.SemaphoreType.DMA] * 17,
)
def _sc_agg(xq0_hbm, xq1_hbm, xq2_hbm, xq3_hbm, src_hbm, dst_hbm,
            zrow_hbm, zdeg_hbm, ones_hbm,
            agg0_hbm, agg1_hbm, agg2_hbm, agg3_hbm, dega_hbm, degb_hbm,
            src_v, dst_v, a0, a1, a2, a3, b0, b1, b2, b3, ones_v,
            acc_sh, deg_sh,
            ga0, ga1, ga2, ga3, gb0, gb1, gb2, gb3,
            sa0, sa1, sa2, sa3, sb0, sb1, sb2, sb3, dsem):
    c = lax.axis_index("c")
    s = lax.axis_index("s")
    rows = pl.ds(s * RPW, RPW)
    bufs = [[a0, a1, a2, a3], [b0, b1, b2, b3]]
    gsems = [[ga0, ga1, ga2, ga3], [gb0, gb1, gb2, gb3]]
    ssems = [[sa0, sa1, sa2, sa3], [sb0, sb1, sb2, sb3]]

    pltpu.sync_copy(ones_hbm, ones_v)

    def one_pass(xq_hbm, agg_hbm, deg_half, deg_hbm):
        pltpu.sync_copy(src_hbm.at[s], src_v)
        pltpu.sync_copy(dst_hbm.at[s], dst_v)
        pltpu.sync_copy(zrow_hbm, acc_sh.at[rows])
        if deg_half is not None:
            pltpu.sync_copy(zdeg_hbm, deg_sh.at[rows])
        plsc.subcore_barrier()

        hdma = xq_hbm.at[pl.ds(0, CHUNK)]

        for h in range(2):
            for b in range(NBUF):
                pltpu.async_copy(
                    xq_hbm.at[src_v.at[h * NBUF + b]], bufs[h][b],
                    gsems[h][b])

        def scat_half(t, h, base):
            for b in range(NBUF):
                pltpu.make_async_copy(hdma, bufs[h][b], gsems[h][b]).wait()
                pltpu.async_copy(
                    bufs[h][b], acc_sh.at[dst_v.at[base + b]], ssems[h][b],
                    add=True)
            if deg_half is not None:
                @pl.when(deg_half(t))
                def _():
                    for b in range(NBUF):
                        pltpu.async_copy(
                            ones_v, deg_sh.at[dst_v.at[base + b]], dsem,
                            add=True)

        def refill_half(t, h, base):
            for b in range(NBUF):
                pltpu.make_async_copy(hdma, bufs[h][b], ssems[h][b]).wait()

                @pl.when(t < NROUND - 1)
                def _(h=h, b=b, base=base):
                    pltpu.async_copy(
                        xq_hbm.at[src_v.at[base + b]], bufs[h][b],
                        gsems[h][b])

        def rnd(t, carry):
            base = t * 2 * NBUF
            scat_half(t, 0, base)
            scat_half(t, 1, base + NBUF)
            refill_half(t, 0, base + 2 * NBUF)
            refill_half(t, 1, base + 3 * NBUF)
            if deg_half is not None:
                @pl.when(deg_half(t))
                def _():
                    for _b in range(2 * NBUF):
                        pltpu.make_async_copy(ones_hbm, ones_v, dsem).wait()
            return carry

        lax.fori_loop(0, NROUND, rnd, 0)
        plsc.subcore_barrier()
        pltpu.sync_copy(acc_sh.at[rows], agg_hbm.at[rows])
        if deg_half is not None:
            pltpu.sync_copy(deg_sh.at[rows], deg_hbm.at[rows])

    @pl.when(c == 0)
    def _():
        one_pass(xq0_hbm, agg0_hbm, lambda t: t < NROUND // 2, dega_hbm)
        one_pass(xq1_hbm, agg1_hbm, None, None)

    @pl.when(c == 1)
    def _():
        one_pass(xq2_hbm, agg2_hbm, lambda t: t >= NROUND // 2, degb_hbm)
        one_pass(xq3_hbm, agg3_hbm, None, None)


def _combine_body(q0_ref, q1_ref, q2_ref, q3_ref,
                  a0_ref, a1_ref, a2_ref, a3_ref, dega_ref, degb_ref,
                  w2s_ref, w2n_ref, b_ref, o_ref):
    B2 = CMB_BLK // 2
    xt = jnp.concatenate(
        [q0_ref[...], q1_ref[...], q2_ref[...], q3_ref[...]], axis=1)
    ag = jnp.concatenate(
        [a0_ref[...], a1_ref[...], a2_ref[...], a3_ref[...]], axis=1)
    deg = (dega_ref[:, 0:1] + degb_ref[:, 0:1]).reshape(B2, 2, 1)
    inv = 1.0 / jnp.maximum(deg, 1.0)
    iv0 = jnp.broadcast_to(inv[:, 0, :], (B2, Q))
    iv1 = jnp.broadcast_to(inv[:, 1, :], (B2, Q))
    ivrow = jnp.concatenate([iv0, iv1] * 4, axis=1)
    h2 = (jnp.dot(xt, w2s_ref[...], preferred_element_type=jnp.float32)
          + jnp.dot(ag * ivrow, w2n_ref[...],
                    preferred_element_type=jnp.float32)
          + b_ref[...])
    he = h2[:, :D]
    ho = h2[:, D:]

    def emap(h):
        nrm = jnp.sqrt(jnp.sum(h * h, axis=1, keepdims=True))
        nrm = jnp.maximum(nrm, EPS)
        return jnp.tanh(nrm) * h / nrm

    oe = emap(he)
    oo = emap(ho)
    o_ref[...] = jnp.stack([oe, oo], axis=1).reshape(CMB_BLK, D)


_W_SPEC = pl.BlockSpec((2 * D, 2 * D), lambda i: (0, 0))
_DEG_SPEC = pl.BlockSpec((CMB_BLK, 16), lambda i: (i, 0))
_CP_SPEC = pl.BlockSpec((CMB_BLK // 2, 2 * Q), lambda i: (i, 0))

_combine = pl.pallas_call(
    _combine_body,
    grid=(N // CMB_BLK,),
    compiler_params=pltpu.CompilerParams(vmem_limit_bytes=100 * 1024 * 1024),
    in_specs=[
        _CP_SPEC, _CP_SPEC, _CP_SPEC, _CP_SPEC,
        _CP_SPEC, _CP_SPEC, _CP_SPEC, _CP_SPEC,
        _DEG_SPEC, _DEG_SPEC,
        _W_SPEC, _W_SPEC,
        pl.BlockSpec((1, 2 * D), lambda i: (0, 0)),
    ],
    out_specs=pl.BlockSpec((CMB_BLK, D), lambda i: (i, 0)),
    out_shape=jax.ShapeDtypeStruct((N, D), jnp.float32),
)


def kernel(x, edge_index, W_self, b_self, W_neigh, b_neigh):
    src = edge_index[0].astype(jnp.int32)
    dst = edge_index[1].astype(jnp.int32)
    pad = E_PAD - E
    src2 = jnp.concatenate([src, jnp.zeros((pad,), jnp.int32)]).reshape(
        NS, NCHUNK, CHUNK)
    dst2 = jnp.concatenate([dst, jnp.full((pad,), TRASH, jnp.int32)]).reshape(
        NS, NCHUNK, CHUNK)

    q0p, q1p, q2p, q3p = _logmap(x)
    q0, q1, q2, q3 = (q.reshape(N, Q) for q in (q0p, q1p, q2p, q3p))

    zrow = jnp.zeros((RPW, Q), jnp.float32)
    zdeg = jnp.zeros((RPW, 16), jnp.float32)
    ones = jnp.ones((CHUNK, 16), jnp.float32)
    a0, a1, a2, a3, dega, degb = _sc_agg(
        q0, q1, q2, q3, src2, dst2, zrow, zdeg, ones)

    a0p, a1p, a2p, a3p = (
        a.reshape(NPAD // 2, 2 * Q) for a in (a0, a1, a2, a3))

    def blockdiag(w):
        z = jnp.zeros((Q, D), jnp.float32)
        blks = []
        for q in range(4):
            wq = w[:, q * Q:(q + 1) * Q].T
            blks.append(jnp.concatenate([wq, z], axis=1))
            blks.append(jnp.concatenate([z, wq], axis=1))
        return jnp.concatenate(blks, axis=0)

    w2s = blockdiag(W_self)
    w2n = blockdiag(W_neigh)
    bias = jnp.tile((b_self + b_neigh), 2).reshape(1, 2 * D)
    return _combine(q0p, q1p, q2p, q3p, a0p, a1p, a2p, a3p, dega, degb,
                    w2s, w2n, bias)


# speedup vs baseline: 1.5406x; 1.3512x over previous
import functools
import jax
import jax.numpy as jnp
from jax import lax
from jax.experimental import pallas as pl
from jax.experimental.pallas import tpu as pltpu
from jax.experimental.pallas import tpu_sc as plsc

N = 10000
D = 256
H = 128
E = 160000
EPS = 1e-7

NS = 16
CHUNK = 64
NCHUNK = 160
NBUF = 4
NROUND = NCHUNK // (2 * NBUF)
E_PAD = NS * NCHUNK * CHUNK
TRASH = N
NPAD = 10112
RPW = NPAD // NS

ROW_BLK = 2000
CMB_BLK = 2000


def _logmap_body(x_ref, h0_ref, h1_ref):
    x = x_ref[...]
    nrm = jnp.sqrt(jnp.sum(x * x, axis=1, keepdims=True))
    nrm = jnp.maximum(nrm, EPS)
    y = jnp.minimum(nrm, 1.0 - 1e-5)
    scale = 0.5 * jnp.log((1.0 + y) / (1.0 - y)) / nrm
    xt = x * scale
    h0_ref[...] = xt[:, :H].astype(jnp.bfloat16)
    h1_ref[...] = xt[:, H:].astype(jnp.bfloat16)


_H_SPEC = pl.BlockSpec((ROW_BLK, H), lambda i: (i, 0))
_H_SHAPE = jax.ShapeDtypeStruct((N, H), jnp.bfloat16)

_logmap = pl.pallas_call(
    _logmap_body,
    grid=(N // ROW_BLK,),
    in_specs=[pl.BlockSpec((ROW_BLK, D), lambda i: (i, 0))],
    out_specs=[_H_SPEC, _H_SPEC],
    out_shape=[_H_SHAPE, _H_SHAPE],
)


@functools.partial(
    pl.kernel,
    mesh=plsc.VectorSubcoreMesh(core_axis_name="c", subcore_axis_name="s"),
    compiler_params=pltpu.CompilerParams(use_tc_tiling_on_sc=False),
    out_type=[
        jax.ShapeDtypeStruct((NPAD, H), jnp.bfloat16),
        jax.ShapeDtypeStruct((NPAD, H), jnp.bfloat16),
        jax.ShapeDtypeStruct((NPAD, 16), jnp.float32),
        jax.ShapeDtypeStruct((NPAD, 16), jnp.float32),
    ],
    scratch_types=[
        pltpu.VMEM((NCHUNK, CHUNK), jnp.int32),
        pltpu.VMEM((NCHUNK, CHUNK), jnp.int32),
        pltpu.VMEM((CHUNK, H), jnp.bfloat16),
        pltpu.VMEM((CHUNK, H), jnp.bfloat16),
        pltpu.VMEM((CHUNK, H), jnp.bfloat16),
        pltpu.VMEM((CHUNK, H), jnp.bfloat16),
        pltpu.VMEM((CHUNK, H), jnp.bfloat16),
        pltpu.VMEM((CHUNK, H), jnp.bfloat16),
        pltpu.VMEM((CHUNK, H), jnp.bfloat16),
        pltpu.VMEM((CHUNK, H), jnp.bfloat16),
        pltpu.VMEM((CHUNK, 16), jnp.float32),
        pltpu.VMEM_SHARED((NPAD, H), jnp.bfloat16),
        pltpu.VMEM_SHARED((NPAD, 16), jnp.float32),
    ] + [pltpu.SemaphoreType.DMA] * 17,
)
def _sc_agg(xh0_hbm, xh1_hbm, src_hbm, dst_hbm,
            zrow_hbm, zdeg_hbm, ones_hbm,
            agg0_hbm, agg1_hbm, dega_hbm, degb_hbm,
            src_v, dst_v, a0, a1, a2, a3, b0, b1, b2, b3, ones_v,
            acc_sh, deg_sh,
            ga0, ga1, ga2, ga3, gb0, gb1, gb2, gb3,
            sa0, sa1, sa2, sa3, sb0, sb1, sb2, sb3, dsem):
    c = lax.axis_index("c")
    s = lax.axis_index("s")
    rows = pl.ds(s * RPW, RPW)
    bufs = [[a0, a1, a2, a3], [b0, b1, b2, b3]]
    gsems = [[ga0, ga1, ga2, ga3], [gb0, gb1, gb2, gb3]]
    ssems = [[sa0, sa1, sa2, sa3], [sb0, sb1, sb2, sb3]]

    pltpu.sync_copy(ones_hbm, ones_v)
    pltpu.sync_copy(src_hbm.at[s], src_v)
    pltpu.sync_copy(dst_hbm.at[s], dst_v)

    def one_pass(xq_hbm, agg_hbm, deg_half, deg_hbm):
        pltpu.sync_copy(zrow_hbm, acc_sh.at[rows])
        pltpu.sync_copy(zdeg_hbm, deg_sh.at[rows])
        plsc.subcore_barrier()

        hdma = xq_hbm.at[pl.ds(0, CHUNK)]

        for h in range(2):
            for b in range(NBUF):
                pltpu.async_copy(
                    xq_hbm.at[src_v.at[h * NBUF + b]], bufs[h][b],
                    gsems[h][b])

        def scat_half(t, h, base):
            for b in range(NBUF):
                pltpu.make_async_copy(hdma, bufs[h][b], gsems[h][b]).wait()
                pltpu.async_copy(
                    bufs[h][b], acc_sh.at[dst_v.at[base + b]], ssems[h][b],
                    add=True)

            @pl.when(deg_half(t))
            def _():
                for b in range(NBUF):
                    pltpu.async_copy(
                        ones_v, deg_sh.at[dst_v.at[base + b]], dsem,
                        add=True)

        def refill_half(t, h, base):
            for b in range(NBUF):
                pltpu.make_async_copy(hdma, bufs[h][b], ssems[h][b]).wait()

                @pl.when(t < NROUND - 1)
                def _(h=h, b=b, base=base):
                    pltpu.async_copy(
                        xq_hbm.at[src_v.at[base + b]], bufs[h][b],
                        gsems[h][b])

        def rnd(t, carry):
            base = t * 2 * NBUF
            scat_half(t, 0, base)
            scat_half(t, 1, base + NBUF)
            refill_half(t, 0, base + 2 * NBUF)
            refill_half(t, 1, base + 3 * NBUF)

            @pl.when(deg_half(t))
            def _():
                for _b in range(2 * NBUF):
                    pltpu.make_async_copy(ones_hbm, ones_v, dsem).wait()
            return carry

        lax.fori_loop(0, NROUND, rnd, 0)
        plsc.subcore_barrier()
        pltpu.sync_copy(acc_sh.at[rows], agg_hbm.at[rows])
        pltpu.sync_copy(deg_sh.at[rows], deg_hbm.at[rows])

    @pl.when(c == 0)
    def _():
        one_pass(xh0_hbm, agg0_hbm, lambda t: t < NROUND // 2, dega_hbm)

    @pl.when(c == 1)
    def _():
        one_pass(xh1_hbm, agg1_hbm, lambda t: t >= NROUND // 2, degb_hbm)


def _combine_body(x0_ref, x1_ref, a0_ref, a1_ref, dega_ref, degb_ref,
                  ws_ref, wn_ref, b_ref, o_ref):
    deg = dega_ref[:, 0:1] + degb_ref[:, 0:1]
    inv = 1.0 / jnp.maximum(deg, 1.0)
    xt = jnp.concatenate(
        [x0_ref[...], x1_ref[...]], axis=1).astype(jnp.float32)
    ag = jnp.concatenate(
        [a0_ref[...], a1_ref[...]], axis=1).astype(jnp.float32)
    dn = (((1,), (1,)), ((), ()))
    h = (lax.dot_general(xt, ws_ref[...], dn,
                         preferred_element_type=jnp.float32)
         + lax.dot_general(ag * inv, wn_ref[...], dn,
                           preferred_element_type=jnp.float32)
         + b_ref[...])
    nrm = jnp.sqrt(jnp.sum(h * h, axis=1, keepdims=True))
    nrm = jnp.maximum(nrm, EPS)
    o_ref[...] = jnp.tanh(nrm) * h / nrm


_CH_SPEC = pl.BlockSpec((CMB_BLK, H), lambda i: (i, 0))
_W_SPEC = pl.BlockSpec((D, D), lambda i: (0, 0))
_DEG_SPEC = pl.BlockSpec((CMB_BLK, 16), lambda i: (i, 0))

_combine = pl.pallas_call(
    _combine_body,
    grid=(N // CMB_BLK,),
    compiler_params=pltpu.CompilerParams(vmem_limit_bytes=100 * 1024 * 1024),
    in_specs=[
        _CH_SPEC, _CH_SPEC,
        _CH_SPEC, _CH_SPEC,
        _DEG_SPEC, _DEG_SPEC,
        _W_SPEC, _W_SPEC,
        pl.BlockSpec((1, D), lambda i: (0, 0)),
    ],
    out_specs=pl.BlockSpec((CMB_BLK, D), lambda i: (i, 0)),
    out_shape=jax.ShapeDtypeStruct((N, D), jnp.float32),
)


def kernel(x, edge_index, W_self, b_self, W_neigh, b_neigh):
    src = edge_index[0].astype(jnp.int32)
    dst = edge_index[1].astype(jnp.int32)
    pad = E_PAD - E
    src2 = jnp.concatenate([src, jnp.zeros((pad,), jnp.int32)]).reshape(
        NS, NCHUNK, CHUNK)
    dst2 = jnp.concatenate([dst, jnp.full((pad,), TRASH, jnp.int32)]).reshape(
        NS, NCHUNK, CHUNK)

    xh0, xh1 = _logmap(x)

    zrow = jnp.zeros((RPW, H), jnp.bfloat16)
    zdeg = jnp.zeros((RPW, 16), jnp.float32)
    ones = jnp.ones((CHUNK, 16), jnp.float32)
    a0, a1, dega, degb = _sc_agg(xh0, xh1, src2, dst2, zrow, zdeg, ones)

    bias = (b_self + b_neigh).reshape(1, D)
    return _combine(xh0, xh1, a0, a1, dega, degb, W_self, W_neigh, bias)
